# baseline (speedup 1.0000x reference)
import functools

import jax
import jax.numpy as jnp
from jax import lax
from jax.experimental import pallas as pl
from jax.experimental.pallas import tpu as pltpu
from jax.experimental.pallas import tpu_sc as plsc

_BSZ = 32
_TC_BATCH = 16


def _sc_broadcast_kernel(num_prefix: int, hidden: int, bsz: int):
    info = plsc.get_sparse_core_info()
    num_cores, num_subcores = info.num_cores, info.num_subcores
    num_workers = num_cores * num_subcores
    rows_per_w = 8
    workers_per_copy = num_prefix // rows_per_w
    num_groups = num_workers // workers_per_copy
    batches_per_w = bsz // num_groups
    mesh = plsc.VectorSubcoreMesh(core_axis_name="c", subcore_axis_name="s")

    @functools.partial(
        pl.kernel,
        mesh=mesh,
        out_type=jax.ShapeDtypeStruct((bsz, num_prefix, hidden), jnp.float32),
        scratch_types=[
            pltpu.VMEM((rows_per_w, hidden), jnp.float32),
            pltpu.SemaphoreType.DMA,
        ],
    )
    def body(table_hbm, out_hbm, stripe_v, sem):
        cid = lax.axis_index("c")
        sid = lax.axis_index("s")
        wid = cid * num_subcores + sid
        group = wid // workers_per_copy
        row0 = (wid % workers_per_copy) * rows_per_w
        b0 = group * batches_per_w
        pltpu.sync_copy(table_hbm.at[pl.ds(row0, rows_per_w)], stripe_v)
        copies = [
            pltpu.async_copy(
                stripe_v, out_hbm.at[b0 + j, pl.ds(row0, rows_per_w)], sem
            )
            for j in range(batches_per_w)
        ]
        for c in copies:
            c.wait()

    return body


def _tc_body(table_ref, out_ref):
    out_ref[0] = table_ref[...]


def _tc_broadcast(table, bsz: int):
    num_prefix, hidden = table.shape
    return pl.pallas_call(
        _tc_body,
        grid=(bsz,),
        in_specs=[
            pl.BlockSpec((num_prefix, hidden), lambda b: (0, 0)),
        ],
        out_specs=pl.BlockSpec((1, num_prefix, hidden), lambda b: (b, 0, 0)),
        out_shape=jax.ShapeDtypeStruct((bsz, num_prefix, hidden), jnp.float32),
    )(table)


def kernel(bsz, prefix_weight):
    num_prefix, hidden = prefix_weight.shape
    tc_half = _tc_broadcast(prefix_weight, _TC_BATCH)
    sc_half = _sc_broadcast_kernel(num_prefix, hidden, _BSZ - _TC_BATCH)(
        prefix_weight
    )
    return jnp.concatenate([tc_half, sc_half], axis=0)


# speedup vs baseline: 3.2714x; 3.2714x over previous
import functools

import jax
import jax.numpy as jnp
from jax import lax
from jax.experimental import pallas as pl
from jax.experimental.pallas import tpu as pltpu
from jax.experimental.pallas import tpu_sc as plsc

_BSZ = 32
_TC_BATCH = 16


def _sc_broadcast_kernel(num_prefix: int, hidden: int, bsz: int):
    info = plsc.get_sparse_core_info()
    num_cores, num_subcores = info.num_cores, info.num_subcores
    num_workers = num_cores * num_subcores
    rows_per_w = 8
    workers_per_copy = num_prefix // rows_per_w
    num_groups = num_workers // workers_per_copy
    batches_per_w = bsz // num_groups
    mesh = plsc.VectorSubcoreMesh(core_axis_name="c", subcore_axis_name="s")

    @functools.partial(
        pl.kernel,
        mesh=mesh,
        out_type=jax.ShapeDtypeStruct((bsz, num_prefix, hidden), jnp.float32),
        scratch_types=[
            pltpu.VMEM((rows_per_w, hidden), jnp.float32),
            pltpu.SemaphoreType.DMA,
        ],
    )
    def body(table_hbm, out_hbm, stripe_v, sem):
        cid = lax.axis_index("c")
        sid = lax.axis_index("s")
        wid = cid * num_subcores + sid
        group = wid // workers_per_copy
        row0 = (wid % workers_per_copy) * rows_per_w
        b0 = group * batches_per_w
        pltpu.sync_copy(table_hbm.at[pl.ds(row0, rows_per_w)], stripe_v)
        copies = [
            pltpu.async_copy(
                stripe_v, out_hbm.at[b0 + j, pl.ds(row0, rows_per_w)], sem
            )
            for j in range(batches_per_w)
        ]
        for c in copies:
            c.wait()

    return body


def _tc_body(table_ref, out_ref):
    out_ref[0] = table_ref[...]


def _tc_broadcast(table, bsz: int):
    num_prefix, hidden = table.shape
    return pl.pallas_call(
        _tc_body,
        grid=(bsz,),
        in_specs=[
            pl.BlockSpec((num_prefix, hidden), lambda b: (0, 0)),
        ],
        out_specs=pl.BlockSpec((1, num_prefix, hidden), lambda b: (b, 0, 0)),
        out_shape=jax.ShapeDtypeStruct((bsz, num_prefix, hidden), jnp.float32),
    )(table)


def kernel(bsz, prefix_weight):
    num_prefix, hidden = prefix_weight.shape
    return _tc_broadcast(prefix_weight, _BSZ)
